# baseline (speedup 1.0000x reference)
import functools

import jax
import jax.numpy as jnp
from jax import lax
from jax.experimental import pallas as pl
from jax.experimental.pallas import tpu as pltpu
from jax.experimental.pallas import tpu_sc as plsc

NUM_CLASSES = 1000000
FEAT_DIM = 32
BATCH = 16384

NC = 2
NS = 16
NW = NC * NS
B_PER_W = BATCH // NW
IDX_CHUNK = 128
N_CHUNKS = B_PER_W // IDX_CHUNK

_mesh = plsc.VectorSubcoreMesh(core_axis_name="c", subcore_axis_name="s")


@functools.partial(
    pl.kernel,
    mesh=_mesh,
    compiler_params=pltpu.CompilerParams(use_tc_tiling_on_sc=False),
    out_type=jax.ShapeDtypeStruct((NW, 16), jnp.float32),
    scratch_types=[
        pltpu.VMEM((N_CHUNKS, IDX_CHUNK), jnp.int32),
        pltpu.VMEM((FEAT_DIM, B_PER_W), jnp.float32),
        pltpu.VMEM((FEAT_DIM, B_PER_W), jnp.float32),
        pltpu.VMEM((16,), jnp.float32),
        pltpu.SemaphoreType.DMA,
    ],
)
def _center_loss_kernel(xt_hbm, labels_hbm, ct_hbm, out_hbm,
                        idx_v, c_v, x_v, acc_v, sem):
    wid = lax.axis_index("s") * NC + lax.axis_index("c")
    base = wid * B_PER_W

    pltpu.sync_copy(labels_hbm.at[wid], idx_v)

    copies = []
    for f in range(FEAT_DIM):
        for j in range(N_CHUNKS):
            copies.append(
                pltpu.async_copy(
                    ct_hbm.at[f].at[idx_v.at[j]],
                    c_v.at[f, pl.ds(j * IDX_CHUNK, IDX_CHUNK)],
                    sem,
                )
            )
    pltpu.sync_copy(xt_hbm.at[:, pl.ds(base, B_PER_W)], x_v)
    for c in copies:
        c.wait()

    def body(g, acc):
        o = g * 16
        for f in range(FEAT_DIM):
            d = x_v[f, pl.ds(o, 16)] - c_v[f, pl.ds(o, 16)]
            acc = acc + d * d
        return acc

    acc = lax.fori_loop(0, B_PER_W // 16, body, jnp.zeros((16,), jnp.float32))
    acc_v[...] = acc
    pltpu.sync_copy(acc_v, out_hbm.at[wid])


def kernel(x, labels, centers):
    labels3 = labels.astype(jnp.int32).reshape(NW, N_CHUNKS, IDX_CHUNK)
    partials = _center_loss_kernel(x.T, labels3, centers.T)
    return jnp.sum(partials)


# speedup vs baseline: 19.9713x; 19.9713x over previous
import functools

import jax
import jax.numpy as jnp
from jax import lax
from jax.experimental import pallas as pl
from jax.experimental.pallas import tpu as pltpu
from jax.experimental.pallas import tpu_sc as plsc

NUM_CLASSES = 1000000
FEAT_DIM = 32
BATCH = 16384

NC = 2
NS = 16
NW = NC * NS
B_PER_W = BATCH // NW
RING = 16
N_GROUPS = B_PER_W // RING

_mesh = plsc.VectorSubcoreMesh(core_axis_name="c", subcore_axis_name="s")


@functools.partial(
    pl.kernel,
    mesh=_mesh,
    compiler_params=pltpu.CompilerParams(needs_layout_passes=False),
    out_type=jax.ShapeDtypeStruct((NW, 8, 128), jnp.float32),
    scratch_types=[
        pltpu.VMEM((4, 128), jnp.int32),
        pltpu.VMEM((FEAT_DIM, B_PER_W), jnp.float32),
        pltpu.VMEM((RING, FEAT_DIM, 128), jnp.float32),
        pltpu.VMEM((8, 128), jnp.float32),
        [pltpu.SemaphoreType.DMA] * RING,
    ],
)
def _center_loss_kernel(xt_hbm, labels_hbm, ct_hbm, out_hbm,
                        idx_v, x_v, blk_v, acc_v, sems):
    wid = lax.axis_index("s") * NC + lax.axis_index("c")
    base = pl.multiple_of(wid * B_PER_W, 128)

    pltpu.sync_copy(labels_hbm.at[wid], idx_v)
    pltpu.sync_copy(xt_hbm.at[:, pl.ds(base, B_PER_W)], x_v)

    iota16 = lax.iota(jnp.int32, 16)
    f_lo = iota16
    f_hi = iota16 + 16
    zeros16 = jnp.zeros((16,), jnp.float32)

    def group_labels(g):
        return idx_v[g // 8, pl.ds((g % 8) * 16, 16)]

    def fire(tv, k):
        t = pl.multiple_of((tv[k] >> 7) << 7, 128)
        pltpu.async_copy(ct_hbm.at[:, pl.ds(t, 128)], blk_v.at[k], sems[k])

    lv0 = group_labels(0)
    for k in range(RING):
        fire(lv0, k)

    def group(g, carry):
        acc, lv = carry
        rem = lax.rem(lv, 128)
        lv_next = group_labels(jnp.minimum(g + 1, N_GROUPS - 1))
        for k in range(RING):
            pltpu.make_async_copy(
                ct_hbm.at[:, pl.ds(0, 128)], blk_v.at[k], sems[k]).wait()
            lane16 = jnp.full((16,), rem[k], jnp.int32)
            k16 = jnp.full((16,), k, jnp.int32)
            i16 = jnp.full((16,), g * RING + k, jnp.int32)
            c_lo = plsc.load_gather(blk_v, [k16, f_lo, lane16])
            c_hi = plsc.load_gather(blk_v, [k16, f_hi, lane16])
            x_lo = plsc.load_gather(x_v, [f_lo, i16])
            x_hi = plsc.load_gather(x_v, [f_hi, i16])
            d_lo = x_lo - c_lo
            d_hi = x_hi - c_hi
            acc = acc + d_lo * d_lo + d_hi * d_hi

            @pl.when(g < N_GROUPS - 1)
            def _():
                fire(lv_next, k)
        return acc, lv_next

    acc, _ = lax.fori_loop(
        0, N_GROUPS, group, (jnp.zeros((16,), jnp.float32), lv0))

    for r in range(8):
        for c in range(0, 128, 16):
            if r == 0 and c == 0:
                continue
            acc_v[r, pl.ds(c, 16)] = zeros16
    acc_v[0, pl.ds(0, 16)] = acc
    pltpu.sync_copy(acc_v, out_hbm.at[wid])


def kernel(x, labels, centers):
    labels3 = labels.astype(jnp.int32).reshape(NW, 4, 128)
    partials = _center_loss_kernel(x.T, labels3, centers.T)
    return jnp.sum(partials)


# speedup vs baseline: 20.1826x; 1.0106x over previous
import functools

import jax
import jax.numpy as jnp
from jax import lax
from jax.experimental import pallas as pl
from jax.experimental.pallas import tpu as pltpu
from jax.experimental.pallas import tpu_sc as plsc

NUM_CLASSES = 1000000
FEAT_DIM = 32
BATCH = 16384

NC = 2
NS = 16
NW = NC * NS
B_PER_W = BATCH // NW
RING = 16
N_GROUPS = B_PER_W // RING

_mesh = plsc.VectorSubcoreMesh(core_axis_name="c", subcore_axis_name="s")


@functools.partial(
    pl.kernel,
    mesh=_mesh,
    compiler_params=pltpu.CompilerParams(needs_layout_passes=False),
    out_type=jax.ShapeDtypeStruct((NW, 8, 128), jnp.float32),
    scratch_types=[
        pltpu.VMEM((4, 128), jnp.int32),
        pltpu.VMEM((FEAT_DIM, B_PER_W), jnp.float32),
        pltpu.VMEM((RING, FEAT_DIM, 128), jnp.float32),
        pltpu.VMEM((8, 128), jnp.float32),
        [pltpu.SemaphoreType.DMA] * RING,
    ],
)
def _center_loss_kernel(xt_hbm, labels_hbm, ct_hbm, out_hbm,
                        idx_v, x_v, blk_v, acc_v, sems):
    wid = lax.axis_index("s") * NC + lax.axis_index("c")
    base = pl.multiple_of(wid * B_PER_W, 128)

    pltpu.sync_copy(labels_hbm.at[wid], idx_v)
    pltpu.sync_copy(xt_hbm.at[:, pl.ds(base, B_PER_W)], x_v)

    iota16 = lax.iota(jnp.int32, 16)
    f_lo = iota16
    f_hi = iota16 + 16
    zeros16 = jnp.zeros((16,), jnp.float32)

    def group_labels(g):
        return idx_v[g // 8, pl.ds((g % 8) * 16, 16)]

    def fire(tv, k):
        t = pl.multiple_of((tv[k] >> 7) << 7, 128)
        for i in range(4):
            pltpu.async_copy(ct_hbm.at[pl.ds(i * 8, 8), pl.ds(t, 128)],
                             blk_v.at[k, pl.ds(i * 8, 8)], sems[k])

    lv0 = group_labels(0)
    for k in range(RING):
        fire(lv0, k)

    def group(g, carry):
        acc, lv = carry
        rem = lax.rem(lv, 128)
        lv_next = group_labels(jnp.minimum(g + 1, N_GROUPS - 1))
        for k in range(RING):
            for i in range(4):
                pltpu.make_async_copy(
                    ct_hbm.at[pl.ds(0, 8), pl.ds(0, 128)],
                    blk_v.at[k, pl.ds(0, 8)], sems[k]).wait()
            lane16 = jnp.full((16,), rem[k], jnp.int32)
            k16 = jnp.full((16,), k, jnp.int32)
            i16 = jnp.full((16,), g * RING + k, jnp.int32)
            c_lo = plsc.load_gather(blk_v, [k16, f_lo, lane16])
            c_hi = plsc.load_gather(blk_v, [k16, f_hi, lane16])
            x_lo = plsc.load_gather(x_v, [f_lo, i16])
            x_hi = plsc.load_gather(x_v, [f_hi, i16])
            d_lo = x_lo - c_lo
            d_hi = x_hi - c_hi
            acc = acc + d_lo * d_lo + d_hi * d_hi

            @pl.when(g < N_GROUPS - 1)
            def _():
                fire(lv_next, k)
        return acc, lv_next

    acc, _ = lax.fori_loop(
        0, N_GROUPS, group, (jnp.zeros((16,), jnp.float32), lv0))

    for r in range(8):
        for c in range(0, 128, 16):
            if r == 0 and c == 0:
                continue
            acc_v[r, pl.ds(c, 16)] = zeros16
    acc_v[0, pl.ds(0, 16)] = acc
    pltpu.sync_copy(acc_v, out_hbm.at[wid])


def kernel(x, labels, centers):
    labels3 = labels.astype(jnp.int32).reshape(NW, 4, 128)
    partials = _center_loss_kernel(x.T, labels3, centers.T)
    return jnp.sum(partials)
